# Initial kernel scaffold; baseline (speedup 1.0000x reference)
import jax
import jax.numpy as jnp
from jax.experimental import pallas as pl


def kernel(pos, neigh_idx, neigh_dist, W1, b1):
    raise NotImplementedError("write your pallas kernel here")


# speedup vs baseline: 3.0193x; 3.0193x over previous
import functools

import jax
import jax.numpy as jnp
from jax import lax
from jax.experimental import pallas as pl
from jax.experimental.pallas import tpu as pltpu
from jax.experimental.pallas import tpu_sc as plsc

B = 2
N = 50000
K = 16
H = 128
NF = 10

NC = 2
NS = 16
NW = NC * NS

CHUNK = 1568
GROUPS = CHUNK // 16

NB = 2000


def _pooled_body(pos_hbm, idx_hbm, dist_hbm, pooled_hbm,
                 plane_v, idx_v, dist_v, pooled_v):
  wid = lax.axis_index("s") * NC + lax.axis_index("c")
  start = jnp.minimum(wid * CHUNK, N - CHUNK).astype(jnp.int32)
  lane = lax.iota(jnp.int32, 16)
  row_k = lane * K
  row_f = lane * NF

  for b in range(B):
    pltpu.sync_copy(idx_hbm.at[pl.ds(b * N * K + start * K, CHUNK * K)], idx_v)
    pltpu.sync_copy(dist_hbm.at[pl.ds(b * N * K + start * K, CHUNK * K)], dist_v)
    for c in range(3):
      pltpu.sync_copy(pos_hbm.at[pl.ds((b * 3 + c) * N, N)], plane_v)

      def group_body(g, carry, c=c):
        gbase = g * (16 * K)
        obase = g * (16 * NF)
        selfv = plane_v[pl.ds(start + g * 16, 16)]
        idx0 = plsc.load_gather(idx_v, [row_k + gbase])
        v = plsc.load_gather(plane_v, [idx0])
        amax = v
        amin = v
        if c == 0:
          dmax = plsc.load_gather(dist_v, [row_k + gbase])
        for k in range(1, K):
          idxk = plsc.load_gather(idx_v, [row_k + (gbase + k)])
          v = plsc.load_gather(plane_v, [idxk])
          amax = jnp.maximum(amax, v)
          amin = jnp.minimum(amin, v)
          if c == 0:
            d = plsc.load_gather(dist_v, [row_k + (gbase + k)])
            dmax = jnp.maximum(dmax, d)
        plsc.store_scatter(pooled_v, [row_f + (obase + c)], selfv)
        plsc.store_scatter(pooled_v, [row_f + (obase + 3 + c)], amax)
        plsc.store_scatter(pooled_v, [row_f + (obase + 6 + c)], selfv - amin)
        if c == 0:
          plsc.store_scatter(pooled_v, [row_f + (obase + 9)], dmax)
        return carry

      lax.fori_loop(0, GROUPS, group_body, 0)
    pltpu.sync_copy(pooled_v, pooled_hbm.at[pl.ds(b * N * NF + start * NF, CHUNK * NF)])


_pooled_call = functools.partial(
    pl.kernel,
    out_type=jax.ShapeDtypeStruct((B * N * NF,), jnp.float32),
    mesh=plsc.VectorSubcoreMesh(core_axis_name="c", subcore_axis_name="s"),
    compiler_params=pltpu.CompilerParams(needs_layout_passes=False),
    scratch_types=[
        pltpu.VMEM((N,), jnp.float32),
        pltpu.VMEM((CHUNK * K,), jnp.int32),
        pltpu.VMEM((CHUNK * K,), jnp.float32),
        pltpu.VMEM((CHUNK * NF,), jnp.float32),
    ],
)(_pooled_body)


def _mlp_body(x_ref, w_ref, b_ref, o_ref):
  x = x_ref[0]
  y = jnp.dot(x, w_ref[...], preferred_element_type=jnp.float32)
  o_ref[0] = jnp.maximum(y + b_ref[...], 0.0)


def _mlp_call(x, wt, b2):
  return pl.pallas_call(
      _mlp_body,
      grid=(B, N // NB),
      in_specs=[
          pl.BlockSpec((1, NB, NF), lambda b, i: (b, i, 0)),
          pl.BlockSpec((NF, H), lambda b, i: (0, 0)),
          pl.BlockSpec((1, H), lambda b, i: (0, 0)),
      ],
      out_specs=pl.BlockSpec((1, NB, H), lambda b, i: (b, i, 0)),
      out_shape=jax.ShapeDtypeStruct((B, N, H), jnp.float32),
  )(x, wt, b2)


def kernel(pos, neigh_idx, neigh_dist, W1, b1):
  pos2 = pos.reshape(B * 3 * N)
  idx_flat = neigh_idx.reshape(B * N * K)
  dist_flat = neigh_dist.reshape(B * N * K)
  pooled = _pooled_call(pos2, idx_flat, dist_flat)
  x = pooled.reshape(B, N, NF)
  return _mlp_call(x, W1.T, b1.reshape(1, H))


# speedup vs baseline: 3.7602x; 1.2454x over previous
import functools

import jax
import jax.numpy as jnp
from jax import lax
from jax.experimental import pallas as pl
from jax.experimental.pallas import tpu as pltpu
from jax.experimental.pallas import tpu_sc as plsc

B = 2
N = 50000
K = 16
H = 128
NF = 10

NC = 2
NS = 16
NW = NC * NS

CHUNK = 1568
GROUPS = CHUNK // 16

NPAD = 51200
NB = 2048
NBLK = NPAD // NB
KP = K // 2


def _prep_body(*refs):
  idx_refs = refs[0:B]
  dist_refs = refs[B:2 * B]
  idxp_refs = refs[2 * B:2 * B + B * KP]
  dm_ref = refs[2 * B + B * KP]
  for b in range(B):
    xt = jnp.transpose(idx_refs[b][...])
    w = jnp.bitwise_or(xt[0:KP], jnp.left_shift(xt[KP:K], 16))
    for t in range(KP):
      idxp_refs[b * KP + t][...] = w[t]
    dt = jnp.transpose(dist_refs[b][...])
    dm_ref[b, 0, :] = jnp.max(dt, axis=0)


def _prep_call(neigh_idx, neigh_dist):
  blk_in = pl.BlockSpec((NB, K), lambda i: (i, 0))
  blk_row = pl.BlockSpec((NB,), lambda i: (i,))
  return pl.pallas_call(
      _prep_body,
      grid=(NBLK,),
      in_specs=[blk_in] * (2 * B),
      out_specs=[blk_row] * (B * KP) + [
          pl.BlockSpec((B, 1, NB), lambda i: (0, 0, i)),
      ],
      out_shape=[jax.ShapeDtypeStruct((NPAD,), jnp.int32)] * (B * KP) + [
          jax.ShapeDtypeStruct((B, 1, NPAD), jnp.float32),
      ],
  )(neigh_idx[0], neigh_idx[1], neigh_dist[0], neigh_dist[1])


def _pooled_body(*refs):
  pos_hbm = refs[0]
  idxp_hbm = refs[1:1 + B * KP]
  pooled_hbm = refs[1 + B * KP]
  plane0_v, plane1_v, idxp_v, pooled_v, sem = refs[2 + B * KP:]
  wid = lax.axis_index("s") * NC + lax.axis_index("c")
  start = jnp.minimum(wid * CHUNK, N - CHUNK).astype(jnp.int32)
  planes = (plane0_v, plane1_v)

  for b in range(B):
    for t in range(KP):
      pltpu.sync_copy(
          idxp_hbm[b * KP + t].at[pl.ds(start, CHUNK)],
          idxp_v.at[pl.ds(t * CHUNK, CHUNK)])
    for c in range(3):
      p = b * 3 + c
      plane_v = planes[p % 2]
      pltpu.sync_copy(pos_hbm.at[pl.ds((b * 3 + c) * N, N)], plane_v)

      def group_body(g, carry, plane_v=plane_v, c=c):
        selfv = plane_v[pl.ds(start + g * 16, 16)]
        amax = None
        for t in range(KP):
          w = idxp_v[pl.ds(t * CHUNK + g * 16, 16)]
          ia = jnp.bitwise_and(w, 0xFFFF)
          ib = lax.shift_right_logical(w, 16)
          va = plsc.load_gather(plane_v, [ia])
          vb = plsc.load_gather(plane_v, [ib])
          hi = jnp.maximum(va, vb)
          lo = jnp.minimum(va, vb)
          if t == 0:
            amax, amin = hi, lo
          else:
            amax = jnp.maximum(amax, hi)
            amin = jnp.minimum(amin, lo)
        pooled_v[pl.ds(c * CHUNK + g * 16, 16)] = selfv
        pooled_v[pl.ds((3 + c) * CHUNK + g * 16, 16)] = amax
        pooled_v[pl.ds((6 + c) * CHUNK + g * 16, 16)] = selfv - amin
        return carry

      lax.fori_loop(0, GROUPS, group_body, 0)
    for f in range(9):
      pltpu.sync_copy(
          pooled_v.at[pl.ds(f * CHUNK, CHUNK)],
          pooled_hbm.at[pl.ds(f * (B * NPAD) + b * NPAD + start, CHUNK)])


_pooled_call = functools.partial(
    pl.kernel,
    out_type=jax.ShapeDtypeStruct((9 * B * NPAD,), jnp.float32),
    mesh=plsc.VectorSubcoreMesh(core_axis_name="c", subcore_axis_name="s"),
    compiler_params=pltpu.CompilerParams(needs_layout_passes=False),
    scratch_types=[
        pltpu.VMEM((N,), jnp.float32),
        pltpu.VMEM((N,), jnp.float32),
        pltpu.VMEM((KP * CHUNK,), jnp.int32),
        pltpu.VMEM((9 * CHUNK,), jnp.float32),
        pltpu.SemaphoreType.DMA,
    ],
)(_pooled_body)


def _mlp_body(x_ref, dm_ref, w_ref, b_ref, o_ref):
  x10 = jnp.concatenate([x_ref[...], dm_ref[0]], axis=0)
  y = lax.dot_general(x10, w_ref[...], (((0,), (0,)), ((), ())),
                      preferred_element_type=jnp.float32)
  o_ref[0] = jnp.maximum(y + b_ref[...], 0.0)


def _mlp_call(xt9, dm, wt, b2):
  return pl.pallas_call(
      _mlp_body,
      grid=(B, NBLK),
      in_specs=[
          pl.BlockSpec((9, NB), lambda b, i: (0, b * NBLK + i)),
          pl.BlockSpec((1, 1, NB), lambda b, i: (b, 0, i)),
          pl.BlockSpec((NF, H), lambda b, i: (0, 0)),
          pl.BlockSpec((1, H), lambda b, i: (0, 0)),
      ],
      out_specs=pl.BlockSpec((1, NB, H), lambda b, i: (b, i, 0)),
      out_shape=jax.ShapeDtypeStruct((B, N, H), jnp.float32),
  )(xt9, dm, wt, b2)


def kernel(pos, neigh_idx, neigh_dist, W1, b1):
  *idxp, dm = _prep_call(neigh_idx, neigh_dist)
  pooled = _pooled_call(pos.reshape(B * 3 * N), *idxp)
  xt9 = pooled.reshape(9, B * NPAD)
  return _mlp_call(xt9, dm, W1.T, b1.reshape(1, H))


# speedup vs baseline: 3.9680x; 1.0552x over previous
import functools

import jax
import jax.numpy as jnp
from jax import lax
from jax.experimental import pallas as pl
from jax.experimental.pallas import tpu as pltpu
from jax.experimental.pallas import tpu_sc as plsc

B = 2
N = 50000
K = 16
H = 128
NF = 10

NC = 2
NS = 16
NW = NC * NS

CHUNK = 1568
GROUPS = CHUNK // 16

NPAD = 51200
NB = 2048
NBLK = NPAD // NB
KP = K // 2


def _prep_body(*refs):
  idx_refs = refs[0:B]
  dist_refs = refs[B:2 * B]
  idxp_refs = refs[2 * B:2 * B + B * KP]
  dm_ref = refs[2 * B + B * KP]
  for b in range(B):
    xt = jnp.transpose(idx_refs[b][...])
    w = jnp.bitwise_or(xt[0:KP], jnp.left_shift(xt[KP:K], 16))
    for t in range(KP):
      idxp_refs[b * KP + t][...] = w[t]
    dt = jnp.transpose(dist_refs[b][...])
    dm_ref[b, 0, :] = jnp.max(dt, axis=0)


def _prep_call(neigh_idx, neigh_dist):
  blk_in = pl.BlockSpec((NB, K), lambda i: (i, 0))
  blk_row = pl.BlockSpec((NB,), lambda i: (i,))
  return pl.pallas_call(
      _prep_body,
      grid=(NBLK,),
      in_specs=[blk_in] * (2 * B),
      out_specs=[blk_row] * (B * KP) + [
          pl.BlockSpec((B, 1, NB), lambda i: (0, 0, i)),
      ],
      out_shape=[jax.ShapeDtypeStruct((NPAD,), jnp.int32)] * (B * KP) + [
          jax.ShapeDtypeStruct((B, 1, NPAD), jnp.float32),
      ],
  )(neigh_idx[0], neigh_idx[1], neigh_dist[0], neigh_dist[1])


def _pooled_body(*refs):
  pos_hbm = refs[0]
  idxp_hbm = refs[1:1 + B * KP]
  pooled_hbm = refs[1 + B * KP]
  plane0_v, plane1_v, idxp_v, pooled_v, sem = refs[2 + B * KP:]
  wid = lax.axis_index("s") * NC + lax.axis_index("c")
  start = jnp.minimum(wid * CHUNK, N - CHUNK).astype(jnp.int32)
  planes = (plane0_v, plane1_v)

  def plane_dma(p):
    b, c = divmod(p, 3)
    return pltpu.async_copy(
        pos_hbm.at[pl.ds((b * 3 + c) * N, N)], planes[p % 2], sem)

  pending = plane_dma(0)
  for b in range(B):
    for t in range(KP):
      pltpu.sync_copy(
          idxp_hbm[b * KP + t].at[pl.ds(start, CHUNK)],
          idxp_v.at[pl.ds(t * CHUNK, CHUNK)])
    for c in range(3):
      p = b * 3 + c
      plane_v = planes[p % 2]
      pending.wait()
      if p + 1 < 6:
        pending = plane_dma(p + 1)

      def group_body(g, carry, plane_v=plane_v, c=c):
        selfv = plane_v[pl.ds(start + g * 16, 16)]
        amax = None
        for t in range(KP):
          w = idxp_v[pl.ds(t * CHUNK + g * 16, 16)]
          ia = jnp.bitwise_and(w, 0xFFFF)
          ib = lax.shift_right_logical(w, 16)
          va = plsc.load_gather(plane_v, [ia])
          vb = plsc.load_gather(plane_v, [ib])
          hi = jnp.maximum(va, vb)
          lo = jnp.minimum(va, vb)
          if t == 0:
            amax, amin = hi, lo
          else:
            amax = jnp.maximum(amax, hi)
            amin = jnp.minimum(amin, lo)
        pooled_v[pl.ds(c * CHUNK + g * 16, 16)] = selfv
        pooled_v[pl.ds((3 + c) * CHUNK + g * 16, 16)] = amax
        pooled_v[pl.ds((6 + c) * CHUNK + g * 16, 16)] = selfv - amin
        return carry

      lax.fori_loop(0, GROUPS, group_body, 0)
    for f in range(9):
      pltpu.sync_copy(
          pooled_v.at[pl.ds(f * CHUNK, CHUNK)],
          pooled_hbm.at[pl.ds(f * (B * NPAD) + b * NPAD + start, CHUNK)])


_pooled_call = functools.partial(
    pl.kernel,
    out_type=jax.ShapeDtypeStruct((9 * B * NPAD,), jnp.float32),
    mesh=plsc.VectorSubcoreMesh(core_axis_name="c", subcore_axis_name="s"),
    compiler_params=pltpu.CompilerParams(needs_layout_passes=False),
    scratch_types=[
        pltpu.VMEM((N,), jnp.float32),
        pltpu.VMEM((N,), jnp.float32),
        pltpu.VMEM((KP * CHUNK,), jnp.int32),
        pltpu.VMEM((9 * CHUNK,), jnp.float32),
        pltpu.SemaphoreType.DMA,
    ],
)(_pooled_body)


def _mlp_body(x_ref, dm_ref, w_ref, b_ref, o_ref):
  x10 = jnp.concatenate([x_ref[...], dm_ref[0]], axis=0)
  y = lax.dot_general(x10, w_ref[...], (((0,), (0,)), ((), ())),
                      preferred_element_type=jnp.float32)
  o_ref[0] = jnp.maximum(y + b_ref[...], 0.0)


def _mlp_call(xt9, dm, wt, b2):
  return pl.pallas_call(
      _mlp_body,
      grid=(B, NBLK),
      in_specs=[
          pl.BlockSpec((9, NB), lambda b, i: (0, b * NBLK + i)),
          pl.BlockSpec((1, 1, NB), lambda b, i: (b, 0, i)),
          pl.BlockSpec((NF, H), lambda b, i: (0, 0)),
          pl.BlockSpec((1, H), lambda b, i: (0, 0)),
      ],
      out_specs=pl.BlockSpec((1, NB, H), lambda b, i: (b, i, 0)),
      out_shape=jax.ShapeDtypeStruct((B, N, H), jnp.float32),
  )(xt9, dm, wt, b2)


def kernel(pos, neigh_idx, neigh_dist, W1, b1):
  *idxp, dm = _prep_call(neigh_idx, neigh_dist)
  pooled = _pooled_call(pos.reshape(B * 3 * N), *idxp)
  xt9 = pooled.reshape(9, B * NPAD)
  return _mlp_call(xt9, dm, W1.T, b1.reshape(1, H))
